# baseline (speedup 1.0000x reference)
import functools

import jax
import jax.numpy as jnp
from jax import lax
from jax.experimental import pallas as pl
from jax.experimental.pallas import tpu as pltpu
from jax.experimental.pallas import tpu_sc as plsc

_BETA = 0.25
_B = 8192
_D = 1024
_K = 1024
_BLK = 256
_KT = 256
_NH = 2
_BH = _B // _NH


def _dist_argmin(r, cb_ref, nsq_ref, d_s):
    rsq = jnp.sum(r * r, axis=1, keepdims=True)
    for kb in range(_K // _KT):
        sl = slice(kb * _KT, (kb + 1) * _KT)
        xc = lax.dot_general(
            r,
            cb_ref[sl, :],
            (((1,), (1,)), ((), ())),
            preferred_element_type=jnp.float32,
            precision=lax.Precision.DEFAULT,
        )
        d_s[:, sl] = (rsq - 2.0 * xc) + nsq_ref[0, sl]
    d = d_s[...]
    m = jnp.min(d, axis=1)
    jix = lax.broadcasted_iota(jnp.int32, (_BLK, _K), 1)
    idx = jnp.min(
        jnp.where(d == m[:, None], jix, jnp.int32(_K)), axis=1
    ).astype(jnp.int32)
    return idx, m


def _emit_idx(idx_ref, loss_ref, idx, m):
    idx_ref[...] = jnp.stack([idx] * 8, axis=0)
    loss_ref[...] = jnp.full((1, 1, 128), jnp.sum(m), jnp.float32)


def _lvl0_body(x_ref, cb_ref, nsq_ref, idx_ref, loss_ref, d_s):
    idx, m = _dist_argmin(x_ref[...], cb_ref, nsq_ref, d_s)
    _emit_idx(idx_ref, loss_ref, idx, m)


def _lvl1_body(x_ref, xq_ref, cb_ref, nsq_ref, r_ref, oa_ref, idx_ref,
               loss_ref, d_s):
    r = x_ref[...]
    xq = xq_ref[...]
    xqst = r + (xq - r)
    oa_ref[...] = xqst
    rn = r - xqst
    r_ref[...] = rn
    idx, m = _dist_argmin(rn, cb_ref, nsq_ref, d_s)
    _emit_idx(idx_ref, loss_ref, idx, m)


def _lvl2_body(r_in_ref, xq_ref, oa_in_ref, cb_ref, nsq_ref, r_ref, oa_ref,
               idx_ref, loss_ref, d_s):
    r = r_in_ref[...]
    xq = xq_ref[...]
    xqst = r + (xq - r)
    oa_ref[...] = oa_in_ref[...] + xqst
    rn = r - xqst
    r_ref[...] = rn
    idx, m = _dist_argmin(rn, cb_ref, nsq_ref, d_s)
    _emit_idx(idx_ref, loss_ref, idx, m)


def _final_body(r_ref, xq_ref, oa_ref, out_ref):
    r = r_ref[...]
    xq = xq_ref[...]
    out_ref[...] = oa_ref[...] + (r + (xq - r))


def _make_tc(nrows):
    nstep = nrows // _BLK
    row_spec = pl.BlockSpec((_BLK, _D), lambda i: (i, 0))
    cb_spec = pl.BlockSpec((_K, _D), lambda i: (0, 0))
    nsq_spec = pl.BlockSpec((1, _K), lambda i: (0, 0))
    idx_spec = pl.BlockSpec((8, _BLK), lambda i: (0, i))
    loss_spec = pl.BlockSpec((1, 1, 128), lambda i: (i, 0, 0))
    idx_shape = jax.ShapeDtypeStruct((8, nrows), jnp.int32)
    loss_shape = jax.ShapeDtypeStruct((nstep, 1, 128), jnp.float32)
    row_shape = jax.ShapeDtypeStruct((nrows, _D), jnp.float32)
    params = pltpu.CompilerParams(dimension_semantics=("parallel",))
    scratch = [pltpu.VMEM((_BLK, _K), jnp.float32)]

    lvl0 = pl.pallas_call(
        _lvl0_body,
        grid=(nstep,),
        in_specs=[row_spec, cb_spec, nsq_spec],
        out_specs=[idx_spec, loss_spec],
        out_shape=[idx_shape, loss_shape],
        scratch_shapes=scratch,
        compiler_params=params,
    )
    lvl1 = pl.pallas_call(
        _lvl1_body,
        grid=(nstep,),
        in_specs=[row_spec, row_spec, cb_spec, nsq_spec],
        out_specs=[row_spec, row_spec, idx_spec, loss_spec],
        out_shape=[row_shape, row_shape, idx_shape, loss_shape],
        scratch_shapes=scratch,
        compiler_params=params,
    )
    lvl2 = pl.pallas_call(
        _lvl2_body,
        grid=(nstep,),
        in_specs=[row_spec, row_spec, row_spec, cb_spec, nsq_spec],
        out_specs=[row_spec, row_spec, idx_spec, loss_spec],
        out_shape=[row_shape, row_shape, idx_shape, loss_shape],
        scratch_shapes=scratch,
        compiler_params=params,
    )
    final = pl.pallas_call(
        _final_body,
        grid=(nstep,),
        in_specs=[row_spec, row_spec, row_spec],
        out_specs=row_spec,
        out_shape=row_shape,
        compiler_params=params,
    )
    return lvl0, lvl1, lvl2, final


_lvl0, _lvl1, _lvl2, _final = _make_tc(_BH)


_NW = 32
_BPW = _BH // _NW
_GCH = 64


def _sc_gather_body(table_hbm, idx_hbm, out_hbm, idx_v, rows_v, sem):
    wid = lax.axis_index("s") * 2 + lax.axis_index("c")
    base = wid * _BPW
    pltpu.sync_copy(idx_hbm.at[pl.ds(base, _BPW)], idx_v)
    for c in range(_BPW // _GCH):
        pltpu.async_copy(
            table_hbm.at[idx_v.at[pl.ds(c * _GCH, _GCH)]], rows_v, sem
        ).wait()
        pltpu.sync_copy(rows_v, out_hbm.at[pl.ds(base + c * _GCH, _GCH)])


_sc_gather = functools.partial(
    pl.kernel,
    mesh=plsc.VectorSubcoreMesh(core_axis_name="c", subcore_axis_name="s"),
    out_type=jax.ShapeDtypeStruct((_BH, _D), jnp.float32),
    scratch_types=[
        pltpu.VMEM((_BPW,), jnp.int32),
        pltpu.VMEM((_GCH, _D), jnp.float32),
        pltpu.SemaphoreType.DMA,
    ],
)(_sc_gather_body)


def kernel(x, cb0, cb1, cb2):
    nsqs = [
        jnp.sum(cb**2, axis=1)[None, :] for cb in (cb0, cb1, cb2)
    ]

    xs = [x[h * _BH : (h + 1) * _BH] for h in range(_NH)]
    idx0p, loss0, xq0 = [None] * _NH, [None] * _NH, [None] * _NH
    r1, oa0, idx1p, loss1, xq1 = ([None] * _NH for _ in range(5))
    r2, oa1, idx2p, loss2, xq2 = ([None] * _NH for _ in range(5))
    outh = [None] * _NH

    for h in range(_NH):
        idx0p[h], loss0[h] = _lvl0(xs[h], cb0, nsqs[0])
    for h in range(_NH):
        xq0[h] = _sc_gather(cb0, idx0p[h][0])
    for h in range(_NH):
        r1[h], oa0[h], idx1p[h], loss1[h] = _lvl1(xs[h], xq0[h], cb1, nsqs[1])
    for h in range(_NH):
        xq1[h] = _sc_gather(cb1, idx1p[h][0])
    for h in range(_NH):
        r2[h], oa1[h], idx2p[h], loss2[h] = _lvl2(
            r1[h], xq1[h], oa0[h], cb2, nsqs[2]
        )
    for h in range(_NH):
        xq2[h] = _sc_gather(cb2, idx2p[h][0])
    for h in range(_NH):
        outh[h] = _final(r2[h], xq2[h], oa1[h])

    out = jnp.concatenate(outh, axis=0)
    loss_sum = sum(
        jnp.sum(l[h][:, 0, 0])
        for l in (loss0, loss1, loss2)
        for h in range(_NH)
    )
    rq_loss = loss_sum * ((1.0 + _BETA) / (3.0 * _B * _D))
    indices = jnp.stack(
        [
            jnp.concatenate([idx0p[h][0] for h in range(_NH)]),
            jnp.concatenate([idx1p[h][0] for h in range(_NH)]),
            jnp.concatenate([idx2p[h][0] for h in range(_NH)]),
        ],
        axis=1,
    )
    return out, rq_loss, indices


# speedup vs baseline: 1.0615x; 1.0615x over previous
import functools

import jax
import jax.numpy as jnp
from jax import lax
from jax.experimental import pallas as pl
from jax.experimental.pallas import tpu as pltpu
from jax.experimental.pallas import tpu_sc as plsc

_BETA = 0.25
_B = 8192
_D = 1024
_K = 1024
_BLK = 256
_KT = 256
_NH = 1
_BH = _B // _NH


def _dist_argmin(r, cb_ref, nsq_ref, d_s):
    rsq = jnp.sum(r * r, axis=1, keepdims=True)
    for kb in range(_K // _KT):
        sl = slice(kb * _KT, (kb + 1) * _KT)
        xc = lax.dot_general(
            r,
            cb_ref[sl, :],
            (((1,), (1,)), ((), ())),
            preferred_element_type=jnp.float32,
            precision=lax.Precision.DEFAULT,
        )
        d_s[:, sl] = (rsq - 2.0 * xc) + nsq_ref[0, sl]
    d = d_s[...]
    m = jnp.min(d, axis=1)
    jix = lax.broadcasted_iota(jnp.int32, (_BLK, _K), 1)
    idx = jnp.min(
        jnp.where(d == m[:, None], jix, jnp.int32(_K)), axis=1
    ).astype(jnp.int32)
    return idx, m


def _emit_idx(idx_ref, loss_ref, idx, m):
    idx_ref[...] = jnp.stack([idx] * 8, axis=0)
    loss_ref[...] = jnp.full((1, 1, 128), jnp.sum(m), jnp.float32)


def _lvl0_body(x_ref, cb_ref, nsq_ref, idx_ref, loss_ref, d_s):
    idx, m = _dist_argmin(x_ref[...], cb_ref, nsq_ref, d_s)
    _emit_idx(idx_ref, loss_ref, idx, m)


def _lvl1_body(x_ref, xq_ref, cb_ref, nsq_ref, r_ref, oa_ref, idx_ref,
               loss_ref, d_s):
    r = x_ref[...]
    xq = xq_ref[...]
    xqst = r + (xq - r)
    oa_ref[...] = xqst
    rn = r - xqst
    r_ref[...] = rn
    idx, m = _dist_argmin(rn, cb_ref, nsq_ref, d_s)
    _emit_idx(idx_ref, loss_ref, idx, m)


def _lvl2_body(r_in_ref, xq_ref, oa_in_ref, cb_ref, nsq_ref, r_ref, oa_ref,
               idx_ref, loss_ref, d_s):
    r = r_in_ref[...]
    xq = xq_ref[...]
    xqst = r + (xq - r)
    oa_ref[...] = oa_in_ref[...] + xqst
    rn = r - xqst
    r_ref[...] = rn
    idx, m = _dist_argmin(rn, cb_ref, nsq_ref, d_s)
    _emit_idx(idx_ref, loss_ref, idx, m)


def _final_body(r_ref, xq_ref, oa_ref, out_ref):
    r = r_ref[...]
    xq = xq_ref[...]
    out_ref[...] = oa_ref[...] + (r + (xq - r))


def _make_tc(nrows):
    nstep = nrows // _BLK
    row_spec = pl.BlockSpec((_BLK, _D), lambda i: (i, 0))
    cb_spec = pl.BlockSpec((_K, _D), lambda i: (0, 0))
    nsq_spec = pl.BlockSpec((1, _K), lambda i: (0, 0))
    idx_spec = pl.BlockSpec((8, _BLK), lambda i: (0, i))
    loss_spec = pl.BlockSpec((1, 1, 128), lambda i: (i, 0, 0))
    idx_shape = jax.ShapeDtypeStruct((8, nrows), jnp.int32)
    loss_shape = jax.ShapeDtypeStruct((nstep, 1, 128), jnp.float32)
    row_shape = jax.ShapeDtypeStruct((nrows, _D), jnp.float32)
    params = pltpu.CompilerParams(dimension_semantics=("parallel",))
    scratch = [pltpu.VMEM((_BLK, _K), jnp.float32)]

    lvl0 = pl.pallas_call(
        _lvl0_body,
        grid=(nstep,),
        in_specs=[row_spec, cb_spec, nsq_spec],
        out_specs=[idx_spec, loss_spec],
        out_shape=[idx_shape, loss_shape],
        scratch_shapes=scratch,
        compiler_params=params,
    )
    lvl1 = pl.pallas_call(
        _lvl1_body,
        grid=(nstep,),
        in_specs=[row_spec, row_spec, cb_spec, nsq_spec],
        out_specs=[row_spec, row_spec, idx_spec, loss_spec],
        out_shape=[row_shape, row_shape, idx_shape, loss_shape],
        scratch_shapes=scratch,
        compiler_params=params,
    )
    lvl2 = pl.pallas_call(
        _lvl2_body,
        grid=(nstep,),
        in_specs=[row_spec, row_spec, row_spec, cb_spec, nsq_spec],
        out_specs=[row_spec, row_spec, idx_spec, loss_spec],
        out_shape=[row_shape, row_shape, idx_shape, loss_shape],
        scratch_shapes=scratch,
        compiler_params=params,
    )
    final = pl.pallas_call(
        _final_body,
        grid=(nstep,),
        in_specs=[row_spec, row_spec, row_spec],
        out_specs=row_spec,
        out_shape=row_shape,
        compiler_params=params,
    )
    return lvl0, lvl1, lvl2, final


_lvl0, _lvl1, _lvl2, _final = _make_tc(_BH)


_NW = 32
_BPW = _BH // _NW
_GCH = 32


_NCH = _BPW // _GCH


def _sc_gather_body(
    table_hbm, idx_hbm, out_hbm, idx_v, buf0, buf1, sg0, sg1, sw0, sw1
):
    wid = lax.axis_index("s") * 2 + lax.axis_index("c")
    base = wid * _BPW
    pltpu.sync_copy(idx_hbm.at[pl.ds(base, _BPW)], idx_v)
    bufs = (buf0, buf1)
    sgs = (sg0, sg1)
    sws = (sw0, sw1)
    hg = [None] * _NCH
    hw = [None] * _NCH
    hg[0] = pltpu.async_copy(
        table_hbm.at[idx_v.at[pl.ds(0, _GCH)]], bufs[0], sgs[0]
    )
    for c in range(_NCH):
        hg[c].wait()
        if c + 1 < _NCH:
            if c >= 1:
                hw[c - 1].wait()
            hg[c + 1] = pltpu.async_copy(
                table_hbm.at[idx_v.at[pl.ds((c + 1) * _GCH, _GCH)]],
                bufs[(c + 1) % 2],
                sgs[(c + 1) % 2],
            )
        hw[c] = pltpu.async_copy(
            bufs[c % 2], out_hbm.at[pl.ds(base + c * _GCH, _GCH)], sws[c % 2]
        )
    if _NCH >= 2:
        hw[_NCH - 2].wait()
    hw[_NCH - 1].wait()


_sc_gather = functools.partial(
    pl.kernel,
    mesh=plsc.VectorSubcoreMesh(core_axis_name="c", subcore_axis_name="s"),
    out_type=jax.ShapeDtypeStruct((_BH, _D), jnp.float32),
    scratch_types=[
        pltpu.VMEM((_BPW,), jnp.int32),
        pltpu.VMEM((_GCH, _D), jnp.float32),
        pltpu.VMEM((_GCH, _D), jnp.float32),
        pltpu.SemaphoreType.DMA,
        pltpu.SemaphoreType.DMA,
        pltpu.SemaphoreType.DMA,
        pltpu.SemaphoreType.DMA,
    ],
)(_sc_gather_body)


def kernel(x, cb0, cb1, cb2):
    nsqs = [
        jnp.sum(cb**2, axis=1)[None, :] for cb in (cb0, cb1, cb2)
    ]

    xs = [x[h * _BH : (h + 1) * _BH] for h in range(_NH)]
    idx0p, loss0, xq0 = [None] * _NH, [None] * _NH, [None] * _NH
    r1, oa0, idx1p, loss1, xq1 = ([None] * _NH for _ in range(5))
    r2, oa1, idx2p, loss2, xq2 = ([None] * _NH for _ in range(5))
    outh = [None] * _NH

    for h in range(_NH):
        idx0p[h], loss0[h] = _lvl0(xs[h], cb0, nsqs[0])
    for h in range(_NH):
        xq0[h] = _sc_gather(cb0, idx0p[h][0])
    for h in range(_NH):
        r1[h], oa0[h], idx1p[h], loss1[h] = _lvl1(xs[h], xq0[h], cb1, nsqs[1])
    for h in range(_NH):
        xq1[h] = _sc_gather(cb1, idx1p[h][0])
    for h in range(_NH):
        r2[h], oa1[h], idx2p[h], loss2[h] = _lvl2(
            r1[h], xq1[h], oa0[h], cb2, nsqs[2]
        )
    for h in range(_NH):
        xq2[h] = _sc_gather(cb2, idx2p[h][0])
    for h in range(_NH):
        outh[h] = _final(r2[h], xq2[h], oa1[h])

    out = jnp.concatenate(outh, axis=0)
    loss_sum = sum(
        jnp.sum(l[h][:, 0, 0])
        for l in (loss0, loss1, loss2)
        for h in range(_NH)
    )
    rq_loss = loss_sum * ((1.0 + _BETA) / (3.0 * _B * _D))
    indices = jnp.stack(
        [
            jnp.concatenate([idx0p[h][0] for h in range(_NH)]),
            jnp.concatenate([idx1p[h][0] for h in range(_NH)]),
            jnp.concatenate([idx2p[h][0] for h in range(_NH)]),
        ],
        axis=1,
    )
    return out, rq_loss, indices


# speedup vs baseline: 1.1715x; 1.1036x over previous
import jax
import jax.numpy as jnp
from jax.experimental import pallas as pl
from jax.experimental.pallas import tpu as pltpu

_BETA = 0.25
_B = 8192
_D = 1024
_K = 1024
_BLK = 256
_KT = 256
_DT = 256


def _trunc16(v):
    u = jax.lax.bitcast_convert_type(v, jnp.uint32)
    return jax.lax.bitcast_convert_type(
        u & jnp.uint32(0xFFFF0000), jnp.float32
    )


def _dot1p(a, b):
    return jax.lax.dot_general(
        a,
        b,
        (((1,), (0,)), ((), ())),
        preferred_element_type=jnp.float32,
        precision=jax.lax.Precision.DEFAULT,
    )


def _rkm_block(
    x_ref,
    rsq0_ref,
    cb0_ref,
    cb1_ref,
    cb2_ref,
    nsq_ref,
    out_ref,
    idx_ref,
    loss_ref,
    r_s,
    d_s,
):
    x = x_ref[...]
    r_s[...] = x
    idxs = []
    loss_row = jnp.zeros((_BLK,), jnp.float32)
    for lvl, cb_ref in enumerate((cb0_ref, cb1_ref, cb2_ref)):
        r = r_s[...]
        if lvl == 0:
            rsq = rsq0_ref[:, 0:1]
        else:
            rsq = jnp.sum(r * r, axis=1, keepdims=True)
        for kb in range(_K // _KT):
            xc_t = jax.lax.dot_general(
                r,
                cb_ref[kb * _KT : (kb + 1) * _KT, :],
                (((1,), (1,)), ((), ())),
                preferred_element_type=jnp.float32,
                precision=jax.lax.Precision.DEFAULT,
            )
            d_s[:, kb * _KT : (kb + 1) * _KT] = (
                rsq - 2.0 * xc_t
            ) + nsq_ref[lvl, kb * _KT : (kb + 1) * _KT]
        d = d_s[...]
        m = jnp.min(d, axis=1)
        jix = jax.lax.broadcasted_iota(jnp.int32, (_BLK, _K), 1)
        idx = jnp.min(
            jnp.where(d == m[:, None], jix, jnp.int32(_K)), axis=1
        ).astype(jnp.int32)
        loss_row = loss_row + m
        onehot = (
            idx[:, None] == jax.lax.broadcasted_iota(jnp.int32, (1, _K), 1)
        ).astype(jnp.float32)
        for db in range(_D // _DT):
            sl = slice(db * _DT, (db + 1) * _DT)
            cbt = cb_ref[:, sl]
            hi = _trunc16(cbt)
            d1 = cbt - hi
            mid = _trunc16(d1)
            lo = d1 - mid
            xq_t = (
                _dot1p(onehot, hi) + _dot1p(onehot, mid)
            ) + _dot1p(onehot, lo)
            rt = r_s[:, sl]
            xqst = rt + (xq_t - rt)
            if lvl == 0:
                out_ref[:, sl] = xqst
            else:
                out_ref[:, sl] += xqst
            r_s[:, sl] = rt - xqst
        idxs.append(idx)
    idx_ref[...] = jnp.stack(idxs + [idxs[0]] * 5, axis=0)
    loss_ref[...] = jnp.full((1, 1, 128), jnp.sum(loss_row), jnp.float32)


def kernel(x, cb0, cb1, cb2):
    nsq = jnp.stack(
        [
            jnp.sum(cb0**2, axis=1),
            jnp.sum(cb1**2, axis=1),
            jnp.sum(cb2**2, axis=1),
        ],
        axis=0,
    )
    nsq = jnp.concatenate([nsq, jnp.zeros((5, _K), jnp.float32)], axis=0)
    rsq0 = jnp.broadcast_to(
        jnp.sum(x**2, axis=1, keepdims=True), (_B, 128)
    )
    grid = (_B // _BLK,)
    n_steps = _B // _BLK
    out, idxp, loss = pl.pallas_call(
        _rkm_block,
        grid=grid,
        in_specs=[
            pl.BlockSpec((_BLK, _D), lambda i: (i, 0)),
            pl.BlockSpec((_BLK, 128), lambda i: (i, 0)),
            pl.BlockSpec((_K, _D), lambda i: (0, 0)),
            pl.BlockSpec((_K, _D), lambda i: (0, 0)),
            pl.BlockSpec((_K, _D), lambda i: (0, 0)),
            pl.BlockSpec((8, _K), lambda i: (0, 0)),
        ],
        out_specs=[
            pl.BlockSpec((_BLK, _D), lambda i: (i, 0)),
            pl.BlockSpec((8, _BLK), lambda i: (0, i)),
            pl.BlockSpec((1, 1, 128), lambda i: (i, 0, 0)),
        ],
        out_shape=[
            jax.ShapeDtypeStruct((_B, _D), jnp.float32),
            jax.ShapeDtypeStruct((8, _B), jnp.int32),
            jax.ShapeDtypeStruct((n_steps, 1, 128), jnp.float32),
        ],
        scratch_shapes=[
            pltpu.VMEM((_BLK, _D), jnp.float32),
            pltpu.VMEM((_BLK, _K), jnp.float32),
        ],
        compiler_params=pltpu.CompilerParams(
            dimension_semantics=("parallel",),
        ),
    )(x, rsq0, cb0, cb1, cb2, nsq)
    rq_loss = jnp.sum(loss[:, 0, 0]) * ((1.0 + _BETA) / (3.0 * _B * _D))
    indices = idxp[:3, :].T
    return out, rq_loss, indices
